# baseline (speedup 1.0000x reference)
import jax
import jax.numpy as jnp
from jax.experimental import pallas as pl
from jax.experimental.pallas import tpu as pltpu

_EDGE_BLOCK = 16000
_NODE_BLOCK = 1000


def _mlp3(x, w1_ref, b1_ref, w2_ref, b2_ref, w3_ref, b3_ref):
    cast = (lambda v: v.astype(jnp.bfloat16)) if x.dtype == jnp.bfloat16 else (lambda v: v)
    h = jnp.dot(x, w1_ref[...], preferred_element_type=jnp.float32) + b1_ref[...]
    h = cast(jnp.maximum(h, 0.0))
    h = jnp.dot(h, w2_ref[...], preferred_element_type=jnp.float32) + b2_ref[...]
    h = cast(jnp.maximum(h, 0.0))
    return jnp.dot(h, w3_ref[...], preferred_element_type=jnp.float32) + b3_ref[...]


def _graph_indep_kernel(
    ex_ref, ew1, eb1, ew2, eb2, ew3, eb3,
    nx_ref, nw1, nb1, nw2, nb2, nw3, nb3,
    gx_ref, gw1, gb1, gw2, gb2, gw3, gb3,
    eo_ref, no_ref, go_ref,
):
    for s in range(2):
        sl = pl.ds(s * (_EDGE_BLOCK // 2), _EDGE_BLOCK // 2)
        eo_ref[sl, :] = _mlp3(ex_ref[sl, :].astype(jnp.bfloat16), ew1, eb1, ew2, eb2, ew3, eb3)
    no_ref[...] = _mlp3(nx_ref[...], nw1, nb1, nw2, nb2, nw3, nb3)

    @pl.when(pl.program_id(0) == 0)
    def _():
        go_ref[...] = _mlp3(gx_ref[...], gw1, gb1, gw2, gb2, gw3, gb3)


def _prep(x, params, cast_bf16=False):
    w1, b1, w2, b2, w3, b3 = params
    if cast_bf16:
        w1, w2, w3 = (w.astype(jnp.bfloat16) for w in (w1, w2, w3))
    return (x, w1, b1.reshape(1, -1), w2, b2.reshape(1, -1), w3, b3.reshape(1, -1))


@jax.jit
def kernel(nodes, edges, global_attr, node_params, edge_params, global_params):
    n_rows = nodes.shape[0]
    e_rows = edges.shape[0]
    d_out = node_params[-1].shape[0]
    grid = (e_rows // _EDGE_BLOCK,)
    assert n_rows // _NODE_BLOCK == grid[0]

    g = jnp.pad(global_attr, ((0, 7), (0, 0)))

    eargs = _prep(edges, edge_params, cast_bf16=True)
    nargs = _prep(nodes, node_params)
    gargs = _prep(g, global_params)

    whole = lambda a: pl.BlockSpec(a.shape, lambda i: (0,) * a.ndim)
    espec = [pl.BlockSpec((_EDGE_BLOCK, edges.shape[1]), lambda i: (i, 0))]
    espec += [whole(a) for a in eargs[1:]]
    nspec = [pl.BlockSpec((_NODE_BLOCK, nodes.shape[1]), lambda i: (i, 0))]
    nspec += [whole(a) for a in nargs[1:]]
    gspec = [whole(a) for a in gargs]

    new_edges, new_nodes, new_global = pl.pallas_call(
        _graph_indep_kernel,
        grid=grid,
        in_specs=espec + nspec + gspec,
        out_specs=[
            pl.BlockSpec((_EDGE_BLOCK, d_out), lambda i: (i, 0)),
            pl.BlockSpec((_NODE_BLOCK, d_out), lambda i: (i, 0)),
            pl.BlockSpec((8, d_out), lambda i: (0, 0)),
        ],
        out_shape=[
            jax.ShapeDtypeStruct((e_rows, d_out), jnp.float32),
            jax.ShapeDtypeStruct((n_rows, d_out), jnp.float32),
            jax.ShapeDtypeStruct((8, d_out), jnp.float32),
        ],
        compiler_params=pltpu.CompilerParams(
            dimension_semantics=("arbitrary",),
        ),
    )(*eargs, *nargs, *gargs)
    return (new_nodes, new_edges, new_global[:1])


# speedup vs baseline: 1.1867x; 1.1867x over previous
import jax
import jax.numpy as jnp
from jax.experimental import pallas as pl
from jax.experimental.pallas import tpu as pltpu

_EDGE_BLOCK = 16000
_NODE_BLOCK = 1000


def _mlp3(x, w1_ref, b1_ref, w2_ref, b2_ref, w3_ref, b3_ref):
    cast = (lambda v: v.astype(jnp.bfloat16)) if x.dtype == jnp.bfloat16 else (lambda v: v)
    h = jnp.dot(x, w1_ref[...], preferred_element_type=jnp.float32) + b1_ref[...]
    h = cast(jnp.maximum(h, 0.0))
    h = jnp.dot(h, w2_ref[...], preferred_element_type=jnp.float32) + b2_ref[...]
    h = cast(jnp.maximum(h, 0.0))
    return jnp.dot(h, w3_ref[...], preferred_element_type=jnp.float32) + b3_ref[...]


def _graph_indep_kernel(
    ex_ref, ew1, eb1, ew2, eb2, ew3, eb3,
    nx_ref, nw1, nb1, nw2, nb2, nw3, nb3,
    gx_ref, gw1, gb1, gw2, gb2, gw3, gb3,
    eo_ref, no_ref, go_ref,
):
    eo_ref[...] = _mlp3(ex_ref[...], ew1, eb1, ew2, eb2, ew3, eb3)
    no_ref[...] = _mlp3(nx_ref[...], nw1, nb1, nw2, nb2, nw3, nb3)

    @pl.when(pl.program_id(0) == 0)
    def _():
        go_ref[...] = _mlp3(gx_ref[...], gw1, gb1, gw2, gb2, gw3, gb3)


def _prep(x, params, cast_bf16=False):
    w1, b1, w2, b2, w3, b3 = params
    if cast_bf16:
        x = x.astype(jnp.bfloat16)
        w1, w2, w3 = (w.astype(jnp.bfloat16) for w in (w1, w2, w3))
    return (x, w1, b1.reshape(1, -1), w2, b2.reshape(1, -1), w3, b3.reshape(1, -1))


@jax.jit
def kernel(nodes, edges, global_attr, node_params, edge_params, global_params):
    n_rows = nodes.shape[0]
    e_rows = edges.shape[0]
    d_out = node_params[-1].shape[0]
    grid = (e_rows // _EDGE_BLOCK,)
    assert n_rows // _NODE_BLOCK == grid[0]

    g = jnp.pad(global_attr, ((0, 7), (0, 0)))

    eargs = _prep(edges, edge_params, cast_bf16=True)
    nargs = _prep(nodes, node_params)
    gargs = _prep(g, global_params)

    whole = lambda a: pl.BlockSpec(a.shape, lambda i: (0,) * a.ndim)
    espec = [pl.BlockSpec((_EDGE_BLOCK, edges.shape[1]), lambda i: (i, 0))]
    espec += [whole(a) for a in eargs[1:]]
    nspec = [pl.BlockSpec((_NODE_BLOCK, nodes.shape[1]), lambda i: (i, 0))]
    nspec += [whole(a) for a in nargs[1:]]
    gspec = [whole(a) for a in gargs]

    new_edges, new_nodes, new_global = pl.pallas_call(
        _graph_indep_kernel,
        grid=grid,
        in_specs=espec + nspec + gspec,
        out_specs=[
            pl.BlockSpec((_EDGE_BLOCK, d_out), lambda i: (i, 0)),
            pl.BlockSpec((_NODE_BLOCK, d_out), lambda i: (i, 0)),
            pl.BlockSpec((8, d_out), lambda i: (0, 0)),
        ],
        out_shape=[
            jax.ShapeDtypeStruct((e_rows, d_out), jnp.float32),
            jax.ShapeDtypeStruct((n_rows, d_out), jnp.float32),
            jax.ShapeDtypeStruct((8, d_out), jnp.float32),
        ],
        compiler_params=pltpu.CompilerParams(
            dimension_semantics=("arbitrary",),
        ),
    )(*eargs, *nargs, *gargs)
    return (new_nodes, new_edges, new_global[:1])


# speedup vs baseline: 1.2159x; 1.0246x over previous
import jax
import jax.numpy as jnp
from jax.experimental import pallas as pl
from jax.experimental.pallas import tpu as pltpu

_EDGE_BLOCK = 16000
_NODE_BLOCK = 1000


def _mlp3(x, w1_ref, b1_ref, w2_ref, b2_ref, w3_ref, b3_ref):
    cast = (lambda v: v.astype(jnp.bfloat16)) if x.dtype == jnp.bfloat16 else (lambda v: v)
    h = jnp.dot(x, w1_ref[...], preferred_element_type=jnp.float32) + b1_ref[...]
    h = cast(jnp.maximum(h, 0.0))
    h = jnp.dot(h, w2_ref[...], preferred_element_type=jnp.float32) + b2_ref[...]
    h = cast(jnp.maximum(h, 0.0))
    return jnp.dot(h, w3_ref[...], preferred_element_type=jnp.float32) + b3_ref[...]


def _graph_indep_kernel(
    ex_ref, ew1, eb1, ew2, eb2, ew3, eb3,
    nx_ref, nw1, nb1, nw2, nb2, nw3, nb3,
    gx_ref, gw1, gb1, gw2, gb2, gw3, gb3,
    eo_ref, no_ref, go_ref,
):
    eo_ref[...] = _mlp3(ex_ref[...], ew1, eb1, ew2, eb2, ew3, eb3)
    no_ref[...] = _mlp3(nx_ref[...], nw1, nb1, nw2, nb2, nw3, nb3)

    @pl.when(pl.program_id(0) == 0)
    def _():
        go_ref[...] = _mlp3(gx_ref[...], gw1, gb1, gw2, gb2, gw3, gb3)


def _prep(x, params, cast_bf16=False):
    w1, b1, w2, b2, w3, b3 = params
    if cast_bf16:
        x = x.astype(jnp.bfloat16)
        w1, w2, w3 = (w.astype(jnp.bfloat16) for w in (w1, w2, w3))
    return (x, w1, b1.reshape(1, -1), w2, b2.reshape(1, -1), w3, b3.reshape(1, -1))


@jax.jit
def kernel(nodes, edges, global_attr, node_params, edge_params, global_params):
    n_rows = nodes.shape[0]
    e_rows = edges.shape[0]
    d_out = node_params[-1].shape[0]
    grid = (e_rows // _EDGE_BLOCK,)
    assert n_rows // _NODE_BLOCK == grid[0]

    eargs = _prep(edges, edge_params, cast_bf16=True)
    nargs = _prep(nodes, node_params)
    gargs = _prep(global_attr, global_params)

    whole = lambda a: pl.BlockSpec(a.shape, lambda i: (0,) * a.ndim)
    espec = [pl.BlockSpec((_EDGE_BLOCK, edges.shape[1]), lambda i: (i, 0))]
    espec += [whole(a) for a in eargs[1:]]
    nspec = [pl.BlockSpec((_NODE_BLOCK, nodes.shape[1]), lambda i: (i, 0))]
    nspec += [whole(a) for a in nargs[1:]]
    gspec = [whole(a) for a in gargs]

    new_edges, new_nodes, new_global = pl.pallas_call(
        _graph_indep_kernel,
        grid=grid,
        in_specs=espec + nspec + gspec,
        out_specs=[
            pl.BlockSpec((_EDGE_BLOCK, d_out), lambda i: (i, 0)),
            pl.BlockSpec((_NODE_BLOCK, d_out), lambda i: (i, 0)),
            pl.BlockSpec((1, d_out), lambda i: (0, 0)),
        ],
        out_shape=[
            jax.ShapeDtypeStruct((e_rows, d_out), jnp.float32),
            jax.ShapeDtypeStruct((n_rows, d_out), jnp.float32),
            jax.ShapeDtypeStruct((1, d_out), jnp.float32),
        ],
        compiler_params=pltpu.CompilerParams(
            dimension_semantics=("arbitrary",),
        ),
    )(*eargs, *nargs, *gargs)
    return (new_nodes, new_edges, new_global)


# speedup vs baseline: 1.2645x; 1.0400x over previous
import jax
import jax.numpy as jnp
from jax.experimental import pallas as pl
from jax.experimental.pallas import tpu as pltpu

_EDGE_BLOCK = 16000
_NODE_BLOCK = 1000


def _mlp3(x, w1_ref, b1_ref, w2_ref, b2_ref, w3_ref, b3_ref):
    cast = (lambda v: v.astype(jnp.bfloat16)) if x.dtype == jnp.bfloat16 else (lambda v: v)
    h = jnp.dot(x, cast(w1_ref[...]), preferred_element_type=jnp.float32) + b1_ref[...]
    h = cast(jnp.maximum(h, 0.0))
    h = jnp.dot(h, cast(w2_ref[...]), preferred_element_type=jnp.float32) + b2_ref[...]
    h = cast(jnp.maximum(h, 0.0))
    return jnp.dot(h, cast(w3_ref[...]), preferred_element_type=jnp.float32) + b3_ref[...]


def _graph_indep_kernel(
    ex_ref, ew1, eb1, ew2, eb2, ew3, eb3,
    nx_ref, nw1, nb1, nw2, nb2, nw3, nb3,
    gx_ref, gw1, gb1, gw2, gb2, gw3, gb3,
    eo_ref, no_ref, go_ref,
):
    eo_ref[...] = _mlp3(ex_ref[...], ew1, eb1, ew2, eb2, ew3, eb3)
    no_ref[...] = _mlp3(nx_ref[...], nw1, nb1, nw2, nb2, nw3, nb3)

    @pl.when(pl.program_id(0) == 0)
    def _():
        go_ref[...] = _mlp3(gx_ref[...], gw1, gb1, gw2, gb2, gw3, gb3)


def _prep(x, params, cast_bf16=False):
    w1, b1, w2, b2, w3, b3 = params
    if cast_bf16:
        x = x.astype(jnp.bfloat16)
    return (x, w1, b1.reshape(1, -1), w2, b2.reshape(1, -1), w3, b3.reshape(1, -1))


@jax.jit
def kernel(nodes, edges, global_attr, node_params, edge_params, global_params):
    n_rows = nodes.shape[0]
    e_rows = edges.shape[0]
    d_out = node_params[-1].shape[0]
    grid = (e_rows // _EDGE_BLOCK,)
    assert n_rows // _NODE_BLOCK == grid[0]

    eargs = _prep(edges, edge_params, cast_bf16=True)
    nargs = _prep(nodes, node_params)
    gargs = _prep(global_attr, global_params)

    whole = lambda a: pl.BlockSpec(a.shape, lambda i: (0,) * a.ndim)
    espec = [pl.BlockSpec((_EDGE_BLOCK, edges.shape[1]), lambda i: (i, 0))]
    espec += [whole(a) for a in eargs[1:]]
    nspec = [pl.BlockSpec((_NODE_BLOCK, nodes.shape[1]), lambda i: (i, 0))]
    nspec += [whole(a) for a in nargs[1:]]
    gspec = [whole(a) for a in gargs]

    new_edges, new_nodes, new_global = pl.pallas_call(
        _graph_indep_kernel,
        grid=grid,
        in_specs=espec + nspec + gspec,
        out_specs=[
            pl.BlockSpec((_EDGE_BLOCK, d_out), lambda i: (i, 0)),
            pl.BlockSpec((_NODE_BLOCK, d_out), lambda i: (i, 0)),
            pl.BlockSpec((1, d_out), lambda i: (0, 0)),
        ],
        out_shape=[
            jax.ShapeDtypeStruct((e_rows, d_out), jnp.float32),
            jax.ShapeDtypeStruct((n_rows, d_out), jnp.float32),
            jax.ShapeDtypeStruct((1, d_out), jnp.float32),
        ],
        compiler_params=pltpu.CompilerParams(
            dimension_semantics=("arbitrary",),
        ),
    )(*eargs, *nargs, *gargs)
    return (new_nodes, new_edges, new_global)
